# Initial kernel scaffold; baseline (speedup 1.0000x reference)
import functools
import jax
import jax.numpy as jnp
from jax import lax
from jax.experimental import pallas as pl
from jax.experimental.pallas import tpu as pltpu

_K = 20
_FH = 240
_FW = 320


def _conv2d(x, w, b=None, pad=0):
    out = lax.conv_general_dilated(
        x, w, (1, 1), [(pad, pad), (pad, pad)],
        dimension_numbers=('NCHW', 'OIHW', 'NCHW'))
    if b is not None:
        out = out + b[None, :, None, None]
    return out


def _bn(x, g, b):
    m = jnp.mean(x, axis=(0, 2, 3), keepdims=True)
    v = jnp.var(x, axis=(0, 2, 3), keepdims=True)
    return (x - m) / jnp.sqrt(v + 1e-5) * g[None, :, None, None] + b[None, :, None, None]


def _lrelu(x):
    return jnp.where(x >= 0, x, 0.2 * x)


def _coord_att(x, w1, b1, g1, bb1, wh, bh, ww, bw):
    B, C, H, W = x.shape
    xh = jnp.mean(x, axis=3, keepdims=True)
    xw = jnp.transpose(jnp.mean(x, axis=2, keepdims=True), (0, 1, 3, 2))
    y = jnp.concatenate([xh, xw], axis=2)
    y = _bn(_conv2d(y, w1, b1), g1, bb1)
    y = y * jnp.clip(y + 3.0, 0.0, 6.0) / 6.0
    yh = y[:, :, :H]
    yw = jnp.transpose(y[:, :, H:], (0, 1, 3, 2))
    ah = jax.nn.sigmoid(_conv2d(yh, wh, bh))
    aw = jax.nn.sigmoid(_conv2d(yw, ww, bw))
    return x * ah * aw


def _edge_body(k, TN, N, C, Cout,
               xt_ref, xf_ref, a_ref, cw_ref,
               premax_ref, stats_ref, pd_ref, r_ref):
    xt = xt_ref[0]
    xf = xf_ref[0]
    ip = lax.dot_general(xt, xf, (((1,), (1,)), ((), ())),
                         preferred_element_type=jnp.float32)
    xx_t = jnp.sum(xt * xt, axis=1, keepdims=True)
    xx_f = jnp.sum(xf * xf, axis=1, keepdims=True).reshape(1, N)
    pd_ref[:] = 2.0 * ip - xx_t - xx_f

    gt = lax.dot_general(xf, a_ref[:], (((1,), (1,)), ((), ())),
                         preferred_element_type=jnp.float32)
    ht = lax.dot_general(xt, cw_ref[:], (((1,), (1,)), ((), ())),
                         preferred_element_type=jnp.float32)

    r_ref[:] = jnp.full((TN, Cout), -jnp.inf, jnp.float32)
    iota = lax.broadcasted_iota(jnp.int32, (TN, N), 1)

    def body(t, carry):
        s1, s2 = carry
        p = pd_ref[:]
        m = jnp.max(p, axis=1, keepdims=True)
        cand = jnp.where(p == m, iota, N)
        jm = jnp.min(cand, axis=1, keepdims=True)
        oh = iota == jm
        pd_ref[:] = jnp.where(oh, -jnp.inf, p)
        g = lax.dot_general(oh.astype(jnp.float32), gt,
                            (((1,), (0,)), ((), ())),
                            preferred_element_type=jnp.float32)
        r_ref[:] = jnp.maximum(r_ref[:], g)
        y = g + ht
        s1 = s1 + jnp.sum(y, axis=0, keepdims=True)
        s2 = s2 + jnp.sum(y * y, axis=0, keepdims=True)
        return s1, s2

    z = jnp.zeros((1, Cout), jnp.float32)
    s1, s2 = lax.fori_loop(0, k, body, (z, z))
    premax_ref[0] = r_ref[:] + ht

    first = jnp.logical_and(pl.program_id(0) == 0, pl.program_id(1) == 0)

    @pl.when(first)
    def _():
        stats_ref[:] = jnp.zeros((8, Cout), jnp.float32)

    pad = jnp.zeros((6, Cout), jnp.float32)
    stats_ref[:] = stats_ref[:] + jnp.concatenate([s1, s2, pad], axis=0)


def _edgeconv(x, w, gamma, beta, k=_K, tn=512):
    B, N, C = x.shape
    Cout = w.shape[0]
    A = w[:, :C]
    Cw = w[:, C:] - w[:, :C]
    NT = N // tn

    premax, stats = pl.pallas_call(
        functools.partial(_edge_body, k, tn, N, C, Cout),
        grid=(B, NT),
        in_specs=[
            pl.BlockSpec((1, tn, C), lambda b, t: (b, t, 0)),
            pl.BlockSpec((1, N, C), lambda b, t: (b, 0, 0)),
            pl.BlockSpec((Cout, C), lambda b, t: (0, 0)),
            pl.BlockSpec((Cout, C), lambda b, t: (0, 0)),
        ],
        out_specs=[
            pl.BlockSpec((1, tn, Cout), lambda b, t: (b, t, 0)),
            pl.BlockSpec((8, Cout), lambda b, t: (0, 0)),
        ],
        out_shape=[
            jax.ShapeDtypeStruct((B, N, Cout), jnp.float32),
            jax.ShapeDtypeStruct((8, Cout), jnp.float32),
        ],
        scratch_shapes=[
            pltpu.VMEM((tn, N), jnp.float32),
            pltpu.VMEM((tn, Cout), jnp.float32),
        ],
    )(x, x, A, Cw)

    cnt = B * N * k
    mean = stats[0] / cnt
    var = stats[1] / cnt - mean * mean
    scale = gamma / jnp.sqrt(var + 1e-5)
    out = premax * scale[None, None, :] + (beta - mean * scale)[None, None, :]
    return jnp.where(out >= 0, out, 0.2 * out)


def _scatter_body(N, HW, pidx_ref, x_ref, out_ref):
    out_ref[:] = jnp.zeros_like(out_ref)

    def body(i, _):
        p = pidx_ref[0, i]
        out_ref[0, pl.ds(p, 1), :] = out_ref[0, pl.ds(p, 1), :] + x_ref[0, pl.ds(i, 1), :]
        return 0

    lax.fori_loop(0, N, body, 0)


def _scatter_add(x, pidx):
    B, N, C = x.shape
    HW = _FH * _FW
    return pl.pallas_call(
        functools.partial(_scatter_body, N, HW),
        grid=(B,),
        in_specs=[
            pl.BlockSpec(memory_space=pltpu.SMEM),
            pl.BlockSpec((1, N, C), lambda b: (b, 0, 0)),
        ],
        out_specs=pl.BlockSpec((1, HW, C), lambda b: (b, 0, 0)),
        out_shape=jax.ShapeDtypeStruct((B, HW, C), jnp.float32),
    )(pidx, x)


def kernel(pc_xyzrgb, feat_s00, pre_w, pre_b, ca1_w, ca1_b, ca_g, ca_b2, cah_w, cah_b, caw_w, caw_b, c1_w, c1_g, c1_b, c2_w, c2_g, c2_b, c3_w, c3_g, c3_b, l4a_w, l4b_w, d5_w, d5_b, d5_g, d5_bb, d6_w, d6_b, d6_g, d6_bb, c7a_w, c7b_w):
    B = pc_xyzrgb.shape[0]
    N = pc_xyzrgb.shape[2]
    v = jnp.floor(pc_xyzrgb[:, 0, :] + 240.0).astype(jnp.int32).reshape(-1)
    u = jnp.floor(pc_xyzrgb[:, 1, :] + 320.0).astype(jnp.int32).reshape(-1)
    bi = jnp.repeat(jnp.arange(B, dtype=jnp.int32), N)
    pc_t = jnp.transpose(pc_xyzrgb, (0, 2, 1))

    f = _conv2d(feat_s00, pre_w, pre_b, pad=1)
    f = _coord_att(f, ca1_w, ca1_b, ca_g, ca_b2, cah_w, cah_b, caw_w, caw_b)
    f = jnp.transpose(f, (0, 2, 3, 1))
    feat_2d = f[bi, v, u].reshape(B, N, 24)

    x0 = jnp.concatenate([pc_t, feat_2d], axis=2)
    x1 = _edgeconv(x0, c1_w.reshape(64, 64), c1_g, c1_b)
    x2 = _edgeconv(x1, c2_w.reshape(64, 128), c2_g, c2_b)
    x3 = _edgeconv(x2, c3_w.reshape(128, 128), c3_g, c3_b)

    x = jnp.concatenate([x1, x2, x3], axis=2)
    x = jnp.matmul(jnp.matmul(x, l4a_w.T), l4b_w.T)

    v2 = v // 2
    u2 = u // 2
    pidx = (v2 * _FW + u2).reshape(B, N)
    fm = _scatter_add(x, pidx)
    fm = jnp.transpose(fm.reshape(B, _FH, _FW, 64), (0, 3, 1, 2))

    fm = jax.nn.softmax(fm, axis=1)
    fm = _lrelu(_bn(_conv2d(fm, d5_w, d5_b, pad=1), d5_g, d5_bb))
    fm = _lrelu(_bn(_conv2d(fm, d6_w, d6_b, pad=1), d6_g, d6_bb))
    fm = _conv2d(_conv2d(fm, c7a_w), c7b_w)
    return (fm, (bi, v2, u2))


# speedup vs baseline: 2.4554x; 2.4554x over previous
import functools
import jax
import jax.numpy as jnp
from jax import lax
from jax.experimental import pallas as pl
from jax.experimental.pallas import tpu as pltpu

_K = 20
_FH = 240
_FW = 320


def _conv2d(x, w, b=None, pad=0):
    out = lax.conv_general_dilated(
        x, w, (1, 1), [(pad, pad), (pad, pad)],
        dimension_numbers=('NCHW', 'OIHW', 'NCHW'))
    if b is not None:
        out = out + b[None, :, None, None]
    return out


def _bn(x, g, b):
    m = jnp.mean(x, axis=(0, 2, 3), keepdims=True)
    v = jnp.var(x, axis=(0, 2, 3), keepdims=True)
    return (x - m) / jnp.sqrt(v + 1e-5) * g[None, :, None, None] + b[None, :, None, None]


def _lrelu(x):
    return jnp.where(x >= 0, x, 0.2 * x)


def _coord_att(x, w1, b1, g1, bb1, wh, bh, ww, bw):
    B, C, H, W = x.shape
    xh = jnp.mean(x, axis=3, keepdims=True)
    xw = jnp.transpose(jnp.mean(x, axis=2, keepdims=True), (0, 1, 3, 2))
    y = jnp.concatenate([xh, xw], axis=2)
    y = _bn(_conv2d(y, w1, b1), g1, bb1)
    y = y * jnp.clip(y + 3.0, 0.0, 6.0) / 6.0
    yh = y[:, :, :H]
    yw = jnp.transpose(y[:, :, H:], (0, 1, 3, 2))
    ah = jax.nn.sigmoid(_conv2d(yh, wh, bh))
    aw = jax.nn.sigmoid(_conv2d(yw, ww, bw))
    return x * ah * aw


def _edge_body(k, TN, N, C, Cout,
               xt_ref, xf_ref, w_ref,
               premax_ref, stats_ref, pd_ref, r_ref):
    xt = xt_ref[0]
    xf = xf_ref[0]
    ip = lax.dot_general(xt.astype(jnp.bfloat16), xf.astype(jnp.bfloat16),
                         (((1,), (1,)), ((), ())),
                         preferred_element_type=jnp.float32)
    xx_t = jnp.sum(xt * xt, axis=1, keepdims=True)
    xx_f = jnp.sum(xf * xf, axis=1, keepdims=True).reshape(1, N)
    pd_ref[:] = (2.0 * ip - xx_t) - xx_f

    wb = w_ref[:].astype(jnp.bfloat16)
    r_ref[:] = jnp.full((TN, Cout), -jnp.inf, jnp.float32)
    iota = lax.broadcasted_iota(jnp.int32, (TN, N), 1)

    def body(t, carry):
        s1, s2 = carry
        p = pd_ref[:]
        m = jnp.max(p, axis=1, keepdims=True)
        cand = jnp.where(p == m, iota, N)
        jm = jnp.min(cand, axis=1, keepdims=True)
        oh = iota == jm
        pd_ref[:] = jnp.where(oh, -jnp.inf, p)
        feat = lax.dot_general(oh.astype(jnp.float32), xf,
                               (((1,), (0,)), ((), ())),
                               preferred_element_type=jnp.float32,
                               precision=lax.Precision.HIGHEST)
        e = jnp.concatenate([feat - xt, xt], axis=1)
        y = lax.dot_general(e.astype(jnp.bfloat16), wb,
                            (((1,), (1,)), ((), ())),
                            preferred_element_type=jnp.float32)
        r_ref[:] = jnp.maximum(r_ref[:], y)
        s1 = s1 + jnp.sum(y, axis=0, keepdims=True)
        s2 = s2 + jnp.sum(y * y, axis=0, keepdims=True)
        return s1, s2

    z = jnp.zeros((1, Cout), jnp.float32)
    s1, s2 = lax.fori_loop(0, k, body, (z, z))
    premax_ref[0] = r_ref[:]

    first = jnp.logical_and(pl.program_id(0) == 0, pl.program_id(1) == 0)

    @pl.when(first)
    def _():
        stats_ref[:] = jnp.zeros((8, Cout), jnp.float32)

    pad = jnp.zeros((6, Cout), jnp.float32)
    stats_ref[:] = stats_ref[:] + jnp.concatenate([s1, s2, pad], axis=0)


def _edgeconv(x, w, gamma, beta, k=_K, tn=512):
    B, N, C = x.shape
    Cout = w.shape[0]
    NT = N // tn

    premax, stats = pl.pallas_call(
        functools.partial(_edge_body, k, tn, N, C, Cout),
        grid=(B, NT),
        in_specs=[
            pl.BlockSpec((1, tn, C), lambda b, t: (b, t, 0)),
            pl.BlockSpec((1, N, C), lambda b, t: (b, 0, 0)),
            pl.BlockSpec((Cout, 2 * C), lambda b, t: (0, 0)),
        ],
        out_specs=[
            pl.BlockSpec((1, tn, Cout), lambda b, t: (b, t, 0)),
            pl.BlockSpec((8, Cout), lambda b, t: (0, 0)),
        ],
        out_shape=[
            jax.ShapeDtypeStruct((B, N, Cout), jnp.float32),
            jax.ShapeDtypeStruct((8, Cout), jnp.float32),
        ],
        scratch_shapes=[
            pltpu.VMEM((tn, N), jnp.float32),
            pltpu.VMEM((tn, Cout), jnp.float32),
        ],
    )(x, x, w)

    cnt = B * N * k
    mean = stats[0] / cnt
    var = stats[1] / cnt - mean * mean
    out = (premax - mean[None, None, :]) / jnp.sqrt(var + 1e-5)[None, None, :] \
        * gamma[None, None, :] + beta[None, None, :]
    return jnp.where(out >= 0, out, 0.2 * out)


def _scatter_body(N, HW2, pidx_ref, x_ref, out_ref):
    out_ref[:] = jnp.zeros(out_ref.shape, out_ref.dtype)
    bb = pl.program_id(0)

    def body(i, _):
        p = pidx_ref[bb, i]
        out_ref[0, pl.ds(p, 1), :] = out_ref[0, pl.ds(p, 1), :] + x_ref[0, pl.ds(i, 1), :]
        return 0

    lax.fori_loop(0, N, body, 0)


def _scatter_add(x, pidx):
    B, N, C = x.shape
    HW = _FH * _FW
    HW2 = HW // 2
    z = jnp.zeros_like(x)
    lo = jnp.concatenate([x, z], axis=2)
    hi = jnp.concatenate([z, x], axis=2)
    x128 = jnp.where((pidx % 2 == 0)[:, :, None], lo, hi)
    r = pidx // 2
    out = pl.pallas_call(
        functools.partial(_scatter_body, N, HW2),
        grid=(B,),
        in_specs=[
            pl.BlockSpec(memory_space=pltpu.SMEM),
            pl.BlockSpec((1, N, 2 * C), lambda b: (b, 0, 0)),
        ],
        out_specs=pl.BlockSpec((1, HW2, 2 * C), lambda b: (b, 0, 0)),
        out_shape=jax.ShapeDtypeStruct((B, HW2, 2 * C), jnp.float32),
    )(r, x128)
    return out.reshape(B, HW, C)


def kernel(pc_xyzrgb, feat_s00, pre_w, pre_b, ca1_w, ca1_b, ca_g, ca_b2, cah_w, cah_b, caw_w, caw_b, c1_w, c1_g, c1_b, c2_w, c2_g, c2_b, c3_w, c3_g, c3_b, l4a_w, l4b_w, d5_w, d5_b, d5_g, d5_bb, d6_w, d6_b, d6_g, d6_bb, c7a_w, c7b_w):
    B = pc_xyzrgb.shape[0]
    N = pc_xyzrgb.shape[2]
    v = jnp.floor(pc_xyzrgb[:, 0, :] + 240.0).astype(jnp.int32).reshape(-1)
    u = jnp.floor(pc_xyzrgb[:, 1, :] + 320.0).astype(jnp.int32).reshape(-1)
    bi = jnp.repeat(jnp.arange(B, dtype=jnp.int32), N)
    pc_t = jnp.transpose(pc_xyzrgb, (0, 2, 1))

    f = _conv2d(feat_s00, pre_w, pre_b, pad=1)
    f = _coord_att(f, ca1_w, ca1_b, ca_g, ca_b2, cah_w, cah_b, caw_w, caw_b)
    f = jnp.transpose(f, (0, 2, 3, 1))
    feat_2d = f[bi, v, u].reshape(B, N, 24)

    x0 = jnp.concatenate([pc_t, feat_2d], axis=2)
    x1 = _edgeconv(x0, c1_w.reshape(64, 64), c1_g, c1_b)
    x2 = _edgeconv(x1, c2_w.reshape(64, 128), c2_g, c2_b)
    x3 = _edgeconv(x2, c3_w.reshape(128, 128), c3_g, c3_b)

    x = jnp.concatenate([x1, x2, x3], axis=2)
    x = jnp.matmul(jnp.matmul(x, l4a_w.T), l4b_w.T)

    v2 = v // 2
    u2 = u // 2
    pidx = (v2 * _FW + u2).reshape(B, N)
    fm = _scatter_add(x, pidx)
    fm = jnp.transpose(fm.reshape(B, _FH, _FW, 64), (0, 3, 1, 2))

    fm = jax.nn.softmax(fm, axis=1)
    fm = _lrelu(_bn(_conv2d(fm, d5_w, d5_b, pad=1), d5_g, d5_bb))
    fm = _lrelu(_bn(_conv2d(fm, d6_w, d6_b, pad=1), d6_g, d6_bb))
    fm = _conv2d(_conv2d(fm, c7a_w), c7b_w)
    return (fm, (bi, v2, u2))


# speedup vs baseline: 5.1089x; 2.0807x over previous
import functools
import jax
import jax.numpy as jnp
from jax import lax
from jax.experimental import pallas as pl
from jax.experimental.pallas import tpu as pltpu
from jax.experimental.pallas import tpu_sc as plsc

_K = 20
_FH = 240
_FW = 320


def _conv2d(x, w, b=None, pad=0):
    out = lax.conv_general_dilated(
        x, w, (1, 1), [(pad, pad), (pad, pad)],
        dimension_numbers=('NCHW', 'OIHW', 'NCHW'))
    if b is not None:
        out = out + b[None, :, None, None]
    return out


def _bn(x, g, b):
    m = jnp.mean(x, axis=(0, 2, 3), keepdims=True)
    v = jnp.var(x, axis=(0, 2, 3), keepdims=True)
    return (x - m) / jnp.sqrt(v + 1e-5) * g[None, :, None, None] + b[None, :, None, None]


def _lrelu(x):
    return jnp.where(x >= 0, x, 0.2 * x)


def _coord_att(x, w1, b1, g1, bb1, wh, bh, ww, bw):
    B, C, H, W = x.shape
    xh = jnp.mean(x, axis=3, keepdims=True)
    xw = jnp.transpose(jnp.mean(x, axis=2, keepdims=True), (0, 1, 3, 2))
    y = jnp.concatenate([xh, xw], axis=2)
    y = _bn(_conv2d(y, w1, b1), g1, bb1)
    y = y * jnp.clip(y + 3.0, 0.0, 6.0) / 6.0
    yh = y[:, :, :H]
    yw = jnp.transpose(y[:, :, H:], (0, 1, 3, 2))
    ah = jax.nn.sigmoid(_conv2d(yh, wh, bh))
    aw = jax.nn.sigmoid(_conv2d(yw, ww, bw))
    return x * ah * aw


def _topk_body(k, TN, N, C, xt_ref, xf_ref, idx_ref, pd_ref):
    xt = xt_ref[0]
    xf = xf_ref[0]
    ip = lax.dot_general(xt.astype(jnp.bfloat16), xf.astype(jnp.bfloat16),
                         (((1,), (1,)), ((), ())),
                         preferred_element_type=jnp.float32)
    xx_t = jnp.sum(xt * xt, axis=1, keepdims=True)
    xx_f = jnp.sum(xf * xf, axis=1, keepdims=True).reshape(1, N)
    pd_ref[:] = (2.0 * ip - xx_t) - xx_f

    iota = lax.broadcasted_iota(jnp.int32, (TN, N), 1)
    liota = lax.broadcasted_iota(jnp.int32, (TN, k), 1)
    base = pl.program_id(0) * N
    acc = jnp.zeros((TN, k), jnp.int32)
    p = pd_ref[:]
    for t in range(k):
        m = jnp.max(p, axis=1, keepdims=True)
        cand = jnp.where(p == m, iota, N)
        jm = jnp.min(cand, axis=1, keepdims=True)
        acc = jnp.where(liota == t, jm + base, acc)
        p = jnp.where(iota == jm, -jnp.inf, p)
    idx_ref[0] = acc


def _topk(x, k=_K, tn=512):
    B, N, C = x.shape
    NT = N // tn
    return pl.pallas_call(
        functools.partial(_topk_body, k, tn, N, C),
        grid=(B, NT),
        in_specs=[
            pl.BlockSpec((1, tn, C), lambda b, t: (b, t, 0)),
            pl.BlockSpec((1, N, C), lambda b, t: (b, 0, 0)),
        ],
        out_specs=pl.BlockSpec((1, tn, k), lambda b, t: (b, t, 0)),
        out_shape=jax.ShapeDtypeStruct((B, N, k), jnp.int32),
        scratch_shapes=[pltpu.VMEM((tn, N), jnp.float32)],
    )(x, x)


_SC_CHUNK = 128


def _sc_gather(x_flat, idx_flat):
    E = idx_flat.shape[0]
    C = x_flat.shape[1]
    info = plsc.get_sparse_core_info()
    nw = info.num_cores * info.num_subcores
    per_w = E // nw
    n_chunks = per_w // _SC_CHUNK
    mesh = plsc.VectorSubcoreMesh(core_axis_name="c", subcore_axis_name="s")

    @functools.partial(
        pl.kernel, mesh=mesh,
        out_type=jax.ShapeDtypeStruct((E, C), jnp.float32),
        scratch_types=[
            pltpu.VMEM((_SC_CHUNK,), jnp.int32),
            pltpu.VMEM((_SC_CHUNK, C), jnp.float32),
            pltpu.SemaphoreType.DMA,
        ],
    )
    def gk(x_hbm, idx_hbm, out_hbm, idx_v, rows_v, sem):
        wid = lax.axis_index("s") * info.num_cores + lax.axis_index("c")
        wbase = wid * per_w

        def body(c, _):
            off = wbase + c * _SC_CHUNK
            pltpu.sync_copy(idx_hbm.at[pl.ds(off, _SC_CHUNK)], idx_v)
            pltpu.async_copy(x_hbm.at[idx_v], rows_v, sem).wait()
            pltpu.sync_copy(rows_v, out_hbm.at[pl.ds(off, _SC_CHUNK)])
            return 0

        lax.fori_loop(0, n_chunks, body, 0)

    return gk(x_flat, idx_flat)


def _econv_body(k, TN, C, Cout, g_ref, xt_ref, w_ref, premax_ref, stats_ref):
    xt = xt_ref[0]
    g3 = g_ref[0][:, :C].reshape(TN, k, C)
    e = g3 - xt[:, None, :]
    xr = jnp.broadcast_to(xt[:, None, :], (TN, k, C))
    ev = jnp.concatenate([e, xr], axis=2).reshape(TN * k, 2 * C)
    y = lax.dot_general(ev.astype(jnp.bfloat16), w_ref[:].astype(jnp.bfloat16),
                        (((1,), (1,)), ((), ())),
                        preferred_element_type=jnp.float32)
    y3 = y.reshape(TN, k, Cout)
    premax_ref[0] = jnp.max(y3, axis=1)
    s1 = jnp.sum(y, axis=0, keepdims=True)
    s2 = jnp.sum(y * y, axis=0, keepdims=True)

    first = jnp.logical_and(pl.program_id(0) == 0, pl.program_id(1) == 0)

    @pl.when(first)
    def _():
        stats_ref[:] = jnp.zeros((8, Cout), jnp.float32)

    pad = jnp.zeros((6, Cout), jnp.float32)
    stats_ref[:] = stats_ref[:] + jnp.concatenate([s1, s2, pad], axis=0)


def _edgeconv(x, w, gamma, beta, k=_K, tn=512):
    B, N, C = x.shape
    Cout = w.shape[0]
    NT = N // tn

    idx = _topk(x, k=k, tn=tn)
    cp = max(128, C)
    xp = jnp.pad(x, ((0, 0), (0, 0), (0, cp - C))) if cp != C else x
    gath = _sc_gather(xp.reshape(B * N, cp), idx.reshape(-1))
    gath = gath.reshape(B, N * k, cp)

    premax, stats = pl.pallas_call(
        functools.partial(_econv_body, k, tn, C, Cout),
        grid=(B, NT),
        in_specs=[
            pl.BlockSpec((1, tn * k, cp), lambda b, t: (b, t, 0)),
            pl.BlockSpec((1, tn, C), lambda b, t: (b, t, 0)),
            pl.BlockSpec((Cout, 2 * C), lambda b, t: (0, 0)),
        ],
        out_specs=[
            pl.BlockSpec((1, tn, Cout), lambda b, t: (b, t, 0)),
            pl.BlockSpec((8, Cout), lambda b, t: (0, 0)),
        ],
        out_shape=[
            jax.ShapeDtypeStruct((B, N, Cout), jnp.float32),
            jax.ShapeDtypeStruct((8, Cout), jnp.float32),
        ],
    )(gath, x, w)

    cnt = B * N * k
    mean = stats[0] / cnt
    var = stats[1] / cnt - mean * mean
    out = (premax - mean[None, None, :]) / jnp.sqrt(var + 1e-5)[None, None, :] \
        * gamma[None, None, :] + beta[None, None, :]
    return jnp.where(out >= 0, out, 0.2 * out)


def _scatter_body(N, HW2, pidx_ref, x_ref, out_ref):
    out_ref[:] = jnp.zeros(out_ref.shape, out_ref.dtype)
    bb = pl.program_id(0)

    def body(i, _):
        p = pidx_ref[bb, i]
        out_ref[0, pl.ds(p, 1), :] = out_ref[0, pl.ds(p, 1), :] + x_ref[0, pl.ds(i, 1), :]
        return 0

    lax.fori_loop(0, N, body, 0)


def _scatter_add(x, pidx):
    B, N, C = x.shape
    HW = _FH * _FW
    HW2 = HW // 2
    z = jnp.zeros_like(x)
    lo = jnp.concatenate([x, z], axis=2)
    hi = jnp.concatenate([z, x], axis=2)
    x128 = jnp.where((pidx % 2 == 0)[:, :, None], lo, hi)
    r = pidx // 2
    out = pl.pallas_call(
        functools.partial(_scatter_body, N, HW2),
        grid=(B,),
        in_specs=[
            pl.BlockSpec(memory_space=pltpu.SMEM),
            pl.BlockSpec((1, N, 2 * C), lambda b: (b, 0, 0)),
        ],
        out_specs=pl.BlockSpec((1, HW2, 2 * C), lambda b: (b, 0, 0)),
        out_shape=jax.ShapeDtypeStruct((B, HW2, 2 * C), jnp.float32),
    )(r, x128)
    return out.reshape(B, HW, C)


def kernel(pc_xyzrgb, feat_s00, pre_w, pre_b, ca1_w, ca1_b, ca_g, ca_b2, cah_w, cah_b, caw_w, caw_b, c1_w, c1_g, c1_b, c2_w, c2_g, c2_b, c3_w, c3_g, c3_b, l4a_w, l4b_w, d5_w, d5_b, d5_g, d5_bb, d6_w, d6_b, d6_g, d6_bb, c7a_w, c7b_w):
    B = pc_xyzrgb.shape[0]
    N = pc_xyzrgb.shape[2]
    v = jnp.floor(pc_xyzrgb[:, 0, :] + 240.0).astype(jnp.int32).reshape(-1)
    u = jnp.floor(pc_xyzrgb[:, 1, :] + 320.0).astype(jnp.int32).reshape(-1)
    bi = jnp.repeat(jnp.arange(B, dtype=jnp.int32), N)
    pc_t = jnp.transpose(pc_xyzrgb, (0, 2, 1))

    f = _conv2d(feat_s00, pre_w, pre_b, pad=1)
    f = _coord_att(f, ca1_w, ca1_b, ca_g, ca_b2, cah_w, cah_b, caw_w, caw_b)
    f = jnp.transpose(f, (0, 2, 3, 1))
    feat_2d = f[bi, v, u].reshape(B, N, 24)

    x0 = jnp.concatenate([pc_t, feat_2d], axis=2)
    x1 = _edgeconv(x0, c1_w.reshape(64, 64), c1_g, c1_b)
    x2 = _edgeconv(x1, c2_w.reshape(64, 128), c2_g, c2_b)
    x3 = _edgeconv(x2, c3_w.reshape(128, 128), c3_g, c3_b)

    x = jnp.concatenate([x1, x2, x3], axis=2)
    x = jnp.matmul(jnp.matmul(x, l4a_w.T), l4b_w.T)

    v2 = v // 2
    u2 = u // 2
    pidx = (v2 * _FW + u2).reshape(B, N)
    fm = _scatter_add(x, pidx)
    fm = jnp.transpose(fm.reshape(B, _FH, _FW, 64), (0, 3, 1, 2))

    fm = jax.nn.softmax(fm, axis=1)
    fm = _lrelu(_bn(_conv2d(fm, d5_w, d5_b, pad=1), d5_g, d5_bb))
    fm = _lrelu(_bn(_conv2d(fm, d6_w, d6_b, pad=1), d6_g, d6_bb))
    fm = _conv2d(_conv2d(fm, c7a_w), c7b_w)
    return (fm, (bi, v2, u2))


# speedup vs baseline: 5.3501x; 1.0472x over previous
import functools
import jax
import jax.numpy as jnp
from jax import lax
from jax.experimental import pallas as pl
from jax.experimental.pallas import tpu as pltpu
from jax.experimental.pallas import tpu_sc as plsc

_K = 20
_FH = 240
_FW = 320


def _conv2d(x, w, b=None, pad=0):
    out = lax.conv_general_dilated(
        x, w, (1, 1), [(pad, pad), (pad, pad)],
        dimension_numbers=('NCHW', 'OIHW', 'NCHW'))
    if b is not None:
        out = out + b[None, :, None, None]
    return out


def _bn(x, g, b):
    m = jnp.mean(x, axis=(0, 2, 3), keepdims=True)
    v = jnp.var(x, axis=(0, 2, 3), keepdims=True)
    return (x - m) / jnp.sqrt(v + 1e-5) * g[None, :, None, None] + b[None, :, None, None]


def _conv_hwc(x, w, b=None, pad=0):
    out = lax.conv_general_dilated(
        x, w, (1, 1), [(pad, pad), (pad, pad)],
        dimension_numbers=('NHWC', 'OIHW', 'NHWC'))
    if b is not None:
        out = out + b[None, None, None, :]
    return out


def _bn_hwc(x, g, b):
    m = jnp.mean(x, axis=(0, 1, 2), keepdims=True)
    v = jnp.var(x, axis=(0, 1, 2), keepdims=True)
    return (x - m) / jnp.sqrt(v + 1e-5) * g[None, None, None, :] + b[None, None, None, :]


def _lrelu(x):
    return jnp.where(x >= 0, x, 0.2 * x)


def _coord_att(x, w1, b1, g1, bb1, wh, bh, ww, bw):
    B, C, H, W = x.shape
    xh = jnp.mean(x, axis=3, keepdims=True)
    xw = jnp.transpose(jnp.mean(x, axis=2, keepdims=True), (0, 1, 3, 2))
    y = jnp.concatenate([xh, xw], axis=2)
    y = _bn(_conv2d(y, w1, b1), g1, bb1)
    y = y * jnp.clip(y + 3.0, 0.0, 6.0) / 6.0
    yh = y[:, :, :H]
    yw = jnp.transpose(y[:, :, H:], (0, 1, 3, 2))
    ah = jax.nn.sigmoid(_conv2d(yh, wh, bh))
    aw = jax.nn.sigmoid(_conv2d(yw, ww, bw))
    return x * ah * aw


def _topk_body(k, TN, N, C, xt_ref, xf_ref, idx_ref, pd_ref):
    xt = xt_ref[0]
    xf = xf_ref[0]
    ip = lax.dot_general(xt.astype(jnp.bfloat16), xf.astype(jnp.bfloat16),
                         (((1,), (1,)), ((), ())),
                         preferred_element_type=jnp.float32)
    xx_t = jnp.sum(xt * xt, axis=1, keepdims=True)
    xx_f = jnp.sum(xf * xf, axis=1, keepdims=True).reshape(1, N)
    pd_ref[:] = (2.0 * ip - xx_t) - xx_f

    iota = lax.broadcasted_iota(jnp.int32, (TN, N), 1)
    liota = lax.broadcasted_iota(jnp.int32, (TN, k), 1)
    base = pl.program_id(0) * N
    acc = jnp.zeros((TN, k), jnp.int32)
    p = pd_ref[:]
    for t in range(k):
        jm = jnp.argmax(p, axis=1).astype(jnp.int32)[:, None]
        acc = jnp.where(liota == t, jm + base, acc)
        p = jnp.where(iota == jm, -jnp.inf, p)
    idx_ref[0] = acc


def _topk(x, k=_K, tn=512):
    B, N, C = x.shape
    NT = N // tn
    return pl.pallas_call(
        functools.partial(_topk_body, k, tn, N, C),
        grid=(B, NT),
        in_specs=[
            pl.BlockSpec((1, tn, C), lambda b, t: (b, t, 0)),
            pl.BlockSpec((1, N, C), lambda b, t: (b, 0, 0)),
        ],
        out_specs=pl.BlockSpec((1, tn, k), lambda b, t: (b, t, 0)),
        out_shape=jax.ShapeDtypeStruct((B, N, k), jnp.int32),
        scratch_shapes=[pltpu.VMEM((tn, N), jnp.float32)],
    )(x, x)


_SC_CHUNK = 128


def _sc_gather(x_flat, idx_flat):
    E = idx_flat.shape[0]
    C = x_flat.shape[1]
    info = plsc.get_sparse_core_info()
    nw = info.num_cores * info.num_subcores
    per_w = E // nw
    n_chunks = per_w // _SC_CHUNK
    mesh = plsc.VectorSubcoreMesh(core_axis_name="c", subcore_axis_name="s")

    @functools.partial(
        pl.kernel, mesh=mesh,
        out_type=jax.ShapeDtypeStruct((E, C), jnp.float32),
        scratch_types=[
            pltpu.VMEM((_SC_CHUNK,), jnp.int32),
            pltpu.VMEM((_SC_CHUNK, C), jnp.float32),
            pltpu.SemaphoreType.DMA,
        ],
    )
    def gk(x_hbm, idx_hbm, out_hbm, idx_v, rows_v, sem):
        wid = lax.axis_index("s") * info.num_cores + lax.axis_index("c")
        wbase = wid * per_w

        def body(c, _):
            off = wbase + c * _SC_CHUNK
            pltpu.sync_copy(idx_hbm.at[pl.ds(off, _SC_CHUNK)], idx_v)
            pltpu.async_copy(x_hbm.at[idx_v], rows_v, sem).wait()
            pltpu.sync_copy(rows_v, out_hbm.at[pl.ds(off, _SC_CHUNK)])
            return 0

        lax.fori_loop(0, n_chunks, body, 0)

    return gk(x_flat, idx_flat)


def _econv_body(k, TN, C, Cout, g_ref, xt_ref, w_ref, premax_ref, stats_ref):
    xt = xt_ref[0]
    g3 = g_ref[0][:, :C].reshape(TN, k, C)
    e = g3 - xt[:, None, :]
    xr = jnp.broadcast_to(xt[:, None, :], (TN, k, C))
    ev = jnp.concatenate([e, xr], axis=2).reshape(TN * k, 2 * C)
    y = lax.dot_general(ev.astype(jnp.bfloat16), w_ref[:].astype(jnp.bfloat16),
                        (((1,), (1,)), ((), ())),
                        preferred_element_type=jnp.float32)
    y3 = y.reshape(TN, k, Cout)
    premax_ref[0] = jnp.max(y3, axis=1)
    s1 = jnp.sum(y, axis=0, keepdims=True)
    s2 = jnp.sum(y * y, axis=0, keepdims=True)

    first = jnp.logical_and(pl.program_id(0) == 0, pl.program_id(1) == 0)

    @pl.when(first)
    def _():
        stats_ref[:] = jnp.zeros((8, Cout), jnp.float32)

    pad = jnp.zeros((6, Cout), jnp.float32)
    stats_ref[:] = stats_ref[:] + jnp.concatenate([s1, s2, pad], axis=0)


def _edgeconv(x, w, gamma, beta, k=_K, tn=512):
    B, N, C = x.shape
    Cout = w.shape[0]
    NT = N // tn

    idx = _topk(x, k=k, tn=tn)
    cp = max(128, C)
    xp = jnp.pad(x, ((0, 0), (0, 0), (0, cp - C))) if cp != C else x
    gath = _sc_gather(xp.reshape(B * N, cp), idx.reshape(-1))
    gath = gath.reshape(B, N * k, cp)

    premax, stats = pl.pallas_call(
        functools.partial(_econv_body, k, tn, C, Cout),
        grid=(B, NT),
        in_specs=[
            pl.BlockSpec((1, tn * k, cp), lambda b, t: (b, t, 0)),
            pl.BlockSpec((1, tn, C), lambda b, t: (b, t, 0)),
            pl.BlockSpec((Cout, 2 * C), lambda b, t: (0, 0)),
        ],
        out_specs=[
            pl.BlockSpec((1, tn, Cout), lambda b, t: (b, t, 0)),
            pl.BlockSpec((8, Cout), lambda b, t: (0, 0)),
        ],
        out_shape=[
            jax.ShapeDtypeStruct((B, N, Cout), jnp.float32),
            jax.ShapeDtypeStruct((8, Cout), jnp.float32),
        ],
    )(gath, x, w)

    cnt = B * N * k
    mean = stats[0] / cnt
    var = stats[1] / cnt - mean * mean
    out = (premax - mean[None, None, :]) / jnp.sqrt(var + 1e-5)[None, None, :] \
        * gamma[None, None, :] + beta[None, None, :]
    return jnp.where(out >= 0, out, 0.2 * out)


def _scatter_body(N, HW2, pidx_ref, x_ref, out_ref):
    out_ref[:] = jnp.zeros(out_ref.shape, out_ref.dtype)
    bb = pl.program_id(0)

    def body(i, _):
        p = pidx_ref[bb, i]
        out_ref[0, pl.ds(p, 1), :] = out_ref[0, pl.ds(p, 1), :] + x_ref[0, pl.ds(i, 1), :]
        return 0

    lax.fori_loop(0, N, body, 0)


def _scatter_add(x, pidx):
    B, N, C = x.shape
    HW = _FH * _FW
    HW2 = HW // 2
    z = jnp.zeros_like(x)
    lo = jnp.concatenate([x, z], axis=2)
    hi = jnp.concatenate([z, x], axis=2)
    x128 = jnp.where((pidx % 2 == 0)[:, :, None], lo, hi)
    r = pidx // 2
    out = pl.pallas_call(
        functools.partial(_scatter_body, N, HW2),
        grid=(B,),
        in_specs=[
            pl.BlockSpec(memory_space=pltpu.SMEM),
            pl.BlockSpec((1, N, 2 * C), lambda b: (b, 0, 0)),
        ],
        out_specs=pl.BlockSpec((1, HW2, 2 * C), lambda b: (b, 0, 0)),
        out_shape=jax.ShapeDtypeStruct((B, HW2, 2 * C), jnp.float32),
    )(r, x128)
    return out.reshape(B, HW, C)


def kernel(pc_xyzrgb, feat_s00, pre_w, pre_b, ca1_w, ca1_b, ca_g, ca_b2, cah_w, cah_b, caw_w, caw_b, c1_w, c1_g, c1_b, c2_w, c2_g, c2_b, c3_w, c3_g, c3_b, l4a_w, l4b_w, d5_w, d5_b, d5_g, d5_bb, d6_w, d6_b, d6_g, d6_bb, c7a_w, c7b_w):
    B = pc_xyzrgb.shape[0]
    N = pc_xyzrgb.shape[2]
    v = jnp.floor(pc_xyzrgb[:, 0, :] + 240.0).astype(jnp.int32).reshape(-1)
    u = jnp.floor(pc_xyzrgb[:, 1, :] + 320.0).astype(jnp.int32).reshape(-1)
    bi = jnp.repeat(jnp.arange(B, dtype=jnp.int32), N)
    pc_t = jnp.transpose(pc_xyzrgb, (0, 2, 1))

    f = _conv2d(feat_s00, pre_w, pre_b, pad=1)
    f = _coord_att(f, ca1_w, ca1_b, ca_g, ca_b2, cah_w, cah_b, caw_w, caw_b)
    f = jnp.transpose(f, (0, 2, 3, 1))
    feat_2d = f[bi, v, u].reshape(B, N, 24)

    x0 = jnp.concatenate([pc_t, feat_2d], axis=2)
    x1 = _edgeconv(x0, c1_w.reshape(64, 64), c1_g, c1_b)
    x2 = _edgeconv(x1, c2_w.reshape(64, 128), c2_g, c2_b)
    x3 = _edgeconv(x2, c3_w.reshape(128, 128), c3_g, c3_b)

    x = jnp.concatenate([x1, x2, x3], axis=2)
    x = jnp.matmul(jnp.matmul(x, l4a_w.T), l4b_w.T)

    v2 = v // 2
    u2 = u // 2
    pidx = (v2 * _FW + u2).reshape(B, N)
    fm = _scatter_add(x, pidx)
    fm = fm.reshape(B, _FH, _FW, 64)

    fm = jax.nn.softmax(fm, axis=3)
    fm = _lrelu(_bn_hwc(_conv_hwc(fm, d5_w, d5_b, pad=1), d5_g, d5_bb))
    fm = _lrelu(_bn_hwc(_conv_hwc(fm, d6_w, d6_b, pad=1), d6_g, d6_bb))
    fm = _conv_hwc(_conv_hwc(fm, c7a_w), c7b_w)
    fm = jnp.transpose(fm, (0, 3, 1, 2))
    return (fm, (bi, v2, u2))


# speedup vs baseline: 5.5005x; 1.0281x over previous
import functools
import jax
import jax.numpy as jnp
from jax import lax
from jax.experimental import pallas as pl
from jax.experimental.pallas import tpu as pltpu
from jax.experimental.pallas import tpu_sc as plsc

_K = 20
_FH = 240
_FW = 320


def _conv2d(x, w, b=None, pad=0):
    out = lax.conv_general_dilated(
        x, w, (1, 1), [(pad, pad), (pad, pad)],
        dimension_numbers=('NCHW', 'OIHW', 'NCHW'))
    if b is not None:
        out = out + b[None, :, None, None]
    return out


def _bn(x, g, b):
    m = jnp.mean(x, axis=(0, 2, 3), keepdims=True)
    v = jnp.var(x, axis=(0, 2, 3), keepdims=True)
    return (x - m) / jnp.sqrt(v + 1e-5) * g[None, :, None, None] + b[None, :, None, None]


def _conv_hwc(x, w, b=None, pad=0):
    out = lax.conv_general_dilated(
        x, w, (1, 1), [(pad, pad), (pad, pad)],
        dimension_numbers=('NHWC', 'OIHW', 'NHWC'))
    if b is not None:
        out = out + b[None, None, None, :]
    return out


def _bn_hwc(x, g, b):
    m = jnp.mean(x, axis=(0, 1, 2), keepdims=True)
    v = jnp.var(x, axis=(0, 1, 2), keepdims=True)
    return (x - m) / jnp.sqrt(v + 1e-5) * g[None, None, None, :] + b[None, None, None, :]


def _coord_att_hwc(x, w1, b1, g1, bb1, wh, bh, ww, bw):
    B, H, W, C = x.shape
    xh = jnp.mean(x, axis=2, keepdims=True)
    xw = jnp.mean(x, axis=1, keepdims=True)
    y = jnp.concatenate([xh, jnp.transpose(xw, (0, 2, 1, 3))], axis=1)
    y = _bn_hwc(_conv_hwc(y, w1, b1), g1, bb1)
    y = y * jnp.clip(y + 3.0, 0.0, 6.0) / 6.0
    yh = y[:, :H]
    yw = jnp.transpose(y[:, H:], (0, 2, 1, 3))
    ah = jax.nn.sigmoid(_conv_hwc(yh, wh, bh))
    aw = jax.nn.sigmoid(_conv_hwc(yw, ww, bw))
    return x * ah * aw


def _lrelu(x):
    return jnp.where(x >= 0, x, 0.2 * x)


def _coord_att(x, w1, b1, g1, bb1, wh, bh, ww, bw):
    B, C, H, W = x.shape
    xh = jnp.mean(x, axis=3, keepdims=True)
    xw = jnp.transpose(jnp.mean(x, axis=2, keepdims=True), (0, 1, 3, 2))
    y = jnp.concatenate([xh, xw], axis=2)
    y = _bn(_conv2d(y, w1, b1), g1, bb1)
    y = y * jnp.clip(y + 3.0, 0.0, 6.0) / 6.0
    yh = y[:, :, :H]
    yw = jnp.transpose(y[:, :, H:], (0, 1, 3, 2))
    ah = jax.nn.sigmoid(_conv2d(yh, wh, bh))
    aw = jax.nn.sigmoid(_conv2d(yw, ww, bw))
    return x * ah * aw


def _topk_body(k, TN, N, C, xt_ref, xf_ref, idx_ref, pd_ref):
    xt = xt_ref[0]
    xf = xf_ref[0]
    ip = lax.dot_general(xt.astype(jnp.bfloat16), xf.astype(jnp.bfloat16),
                         (((1,), (1,)), ((), ())),
                         preferred_element_type=jnp.float32)
    xx_t = jnp.sum(xt * xt, axis=1, keepdims=True)
    xx_f = jnp.sum(xf * xf, axis=1, keepdims=True).reshape(1, N)
    pd_ref[:] = (2.0 * ip - xx_t) - xx_f

    iota = lax.broadcasted_iota(jnp.int32, (TN, N), 1)
    liota = lax.broadcasted_iota(jnp.int32, (TN, k), 1)
    base = pl.program_id(0) * N
    acc = jnp.zeros((TN, k), jnp.int32)
    p = pd_ref[:]
    for t in range(k):
        jm = jnp.argmax(p, axis=1).astype(jnp.int32)[:, None]
        acc = jnp.where(liota == t, jm + base, acc)
        p = jnp.where(iota == jm, -jnp.inf, p)
    idx_ref[0] = acc


def _topk(x, k=_K, tn=512):
    B, N, C = x.shape
    NT = N // tn
    return pl.pallas_call(
        functools.partial(_topk_body, k, tn, N, C),
        grid=(B, NT),
        in_specs=[
            pl.BlockSpec((1, tn, C), lambda b, t: (b, t, 0)),
            pl.BlockSpec((1, N, C), lambda b, t: (b, 0, 0)),
        ],
        out_specs=pl.BlockSpec((1, tn, k), lambda b, t: (b, t, 0)),
        out_shape=jax.ShapeDtypeStruct((B, N, k), jnp.int32),
        scratch_shapes=[pltpu.VMEM((tn, N), jnp.float32)],
    )(x, x)


_SC_CHUNK = 128


def _sc_gather(x_flat, idx_flat):
    E = idx_flat.shape[0]
    C = x_flat.shape[1]
    info = plsc.get_sparse_core_info()
    nw = info.num_cores * info.num_subcores
    per_w = E // nw
    n_chunks = per_w // _SC_CHUNK
    mesh = plsc.VectorSubcoreMesh(core_axis_name="c", subcore_axis_name="s")

    @functools.partial(
        pl.kernel, mesh=mesh,
        out_type=jax.ShapeDtypeStruct((E, C), jnp.float32),
        scratch_types=[
            pltpu.VMEM((_SC_CHUNK,), jnp.int32),
            pltpu.VMEM((_SC_CHUNK, C), jnp.float32),
            pltpu.SemaphoreType.DMA,
        ],
    )
    def gk(x_hbm, idx_hbm, out_hbm, idx_v, rows_v, sem):
        wid = lax.axis_index("s") * info.num_cores + lax.axis_index("c")
        wbase = wid * per_w

        def body(c, _):
            off = wbase + c * _SC_CHUNK
            pltpu.sync_copy(idx_hbm.at[pl.ds(off, _SC_CHUNK)], idx_v)
            pltpu.async_copy(x_hbm.at[idx_v], rows_v, sem).wait()
            pltpu.sync_copy(rows_v, out_hbm.at[pl.ds(off, _SC_CHUNK)])
            return 0

        lax.fori_loop(0, n_chunks, body, 0)

    return gk(x_flat, idx_flat)


def _econv_body(k, TN, C, Cout, g_ref, xt_ref, w_ref, premax_ref, stats_ref):
    xt = xt_ref[0]
    g3 = g_ref[0][:, :C].reshape(TN, k, C)
    e = g3 - xt[:, None, :]
    xr = jnp.broadcast_to(xt[:, None, :], (TN, k, C))
    ev = jnp.concatenate([e, xr], axis=2).reshape(TN * k, 2 * C)
    y = lax.dot_general(ev.astype(jnp.bfloat16), w_ref[:].astype(jnp.bfloat16),
                        (((1,), (1,)), ((), ())),
                        preferred_element_type=jnp.float32)
    y3 = y.reshape(TN, k, Cout)
    premax_ref[0] = jnp.max(y3, axis=1)
    s1 = jnp.sum(y, axis=0, keepdims=True)
    s2 = jnp.sum(y * y, axis=0, keepdims=True)

    first = jnp.logical_and(pl.program_id(0) == 0, pl.program_id(1) == 0)

    @pl.when(first)
    def _():
        stats_ref[:] = jnp.zeros((8, Cout), jnp.float32)

    pad = jnp.zeros((6, Cout), jnp.float32)
    stats_ref[:] = stats_ref[:] + jnp.concatenate([s1, s2, pad], axis=0)


def _edgeconv(x, w, gamma, beta, k=_K, tn=512):
    B, N, C = x.shape
    Cout = w.shape[0]
    NT = N // tn

    idx = _topk(x, k=k, tn=tn)
    cp = max(128, C)
    xp = jnp.pad(x, ((0, 0), (0, 0), (0, cp - C))) if cp != C else x
    gath = _sc_gather(xp.reshape(B * N, cp), idx.reshape(-1))
    gath = gath.reshape(B, N * k, cp)

    premax, stats = pl.pallas_call(
        functools.partial(_econv_body, k, tn, C, Cout),
        grid=(B, NT),
        in_specs=[
            pl.BlockSpec((1, tn * k, cp), lambda b, t: (b, t, 0)),
            pl.BlockSpec((1, tn, C), lambda b, t: (b, t, 0)),
            pl.BlockSpec((Cout, 2 * C), lambda b, t: (0, 0)),
        ],
        out_specs=[
            pl.BlockSpec((1, tn, Cout), lambda b, t: (b, t, 0)),
            pl.BlockSpec((8, Cout), lambda b, t: (0, 0)),
        ],
        out_shape=[
            jax.ShapeDtypeStruct((B, N, Cout), jnp.float32),
            jax.ShapeDtypeStruct((8, Cout), jnp.float32),
        ],
    )(gath, x, w)

    cnt = B * N * k
    mean = stats[0] / cnt
    var = stats[1] / cnt - mean * mean
    out = (premax - mean[None, None, :]) / jnp.sqrt(var + 1e-5)[None, None, :] \
        * gamma[None, None, :] + beta[None, None, :]
    return jnp.where(out >= 0, out, 0.2 * out)


def _scatter_body(N, HW2, pidx_ref, x_ref, out_ref):
    out_ref[:] = jnp.zeros(out_ref.shape, out_ref.dtype)
    bb = pl.program_id(0)

    def body(i, _):
        p = pidx_ref[bb, i]
        out_ref[0, pl.ds(p, 1), :] = out_ref[0, pl.ds(p, 1), :] + x_ref[0, pl.ds(i, 1), :]
        return 0

    lax.fori_loop(0, N, body, 0)


def _scatter_add(x, pidx):
    B, N, C = x.shape
    HW = _FH * _FW
    HW2 = HW // 2
    z = jnp.zeros_like(x)
    lo = jnp.concatenate([x, z], axis=2)
    hi = jnp.concatenate([z, x], axis=2)
    x128 = jnp.where((pidx % 2 == 0)[:, :, None], lo, hi)
    r = pidx // 2
    out = pl.pallas_call(
        functools.partial(_scatter_body, N, HW2),
        grid=(B,),
        in_specs=[
            pl.BlockSpec(memory_space=pltpu.SMEM),
            pl.BlockSpec((1, N, 2 * C), lambda b: (b, 0, 0)),
        ],
        out_specs=pl.BlockSpec((1, HW2, 2 * C), lambda b: (b, 0, 0)),
        out_shape=jax.ShapeDtypeStruct((B, HW2, 2 * C), jnp.float32),
    )(r, x128)
    return out.reshape(B, HW, C)


def kernel(pc_xyzrgb, feat_s00, pre_w, pre_b, ca1_w, ca1_b, ca_g, ca_b2, cah_w, cah_b, caw_w, caw_b, c1_w, c1_g, c1_b, c2_w, c2_g, c2_b, c3_w, c3_g, c3_b, l4a_w, l4b_w, d5_w, d5_b, d5_g, d5_bb, d6_w, d6_b, d6_g, d6_bb, c7a_w, c7b_w):
    B = pc_xyzrgb.shape[0]
    N = pc_xyzrgb.shape[2]
    v = jnp.floor(pc_xyzrgb[:, 0, :] + 240.0).astype(jnp.int32).reshape(-1)
    u = jnp.floor(pc_xyzrgb[:, 1, :] + 320.0).astype(jnp.int32).reshape(-1)
    bi = jnp.repeat(jnp.arange(B, dtype=jnp.int32), N)
    pc_t = jnp.transpose(pc_xyzrgb, (0, 2, 1))

    fs = jnp.transpose(feat_s00, (0, 2, 3, 1))
    f = _conv_hwc(fs, pre_w, pre_b, pad=1)
    f = _coord_att_hwc(f, ca1_w, ca1_b, ca_g, ca_b2, cah_w, cah_b, caw_w, caw_b)
    feat_2d = f[bi, v, u].reshape(B, N, 24)

    x0 = jnp.concatenate([pc_t, feat_2d], axis=2)
    x1 = _edgeconv(x0, c1_w.reshape(64, 64), c1_g, c1_b)
    x2 = _edgeconv(x1, c2_w.reshape(64, 128), c2_g, c2_b)
    x3 = _edgeconv(x2, c3_w.reshape(128, 128), c3_g, c3_b)

    x = jnp.concatenate([x1, x2, x3], axis=2)
    x = jnp.matmul(jnp.matmul(x, l4a_w.T), l4b_w.T)

    v2 = v // 2
    u2 = u // 2
    pidx = (v2 * _FW + u2).reshape(B, N)
    fm = _scatter_add(x, pidx)
    fm = fm.reshape(B, _FH, _FW, 64)

    fm = jax.nn.softmax(fm, axis=3)
    fm = _lrelu(_bn_hwc(_conv_hwc(fm, d5_w, d5_b, pad=1), d5_g, d5_bb))
    fm = _lrelu(_bn_hwc(_conv_hwc(fm, d6_w, d6_b, pad=1), d6_g, d6_bb))
    fm = _conv_hwc(_conv_hwc(fm, c7a_w), c7b_w)
    fm = jnp.transpose(fm, (0, 3, 1, 2))
    return (fm, (bi, v2, u2))


# speedup vs baseline: 5.7407x; 1.0437x over previous
import functools
import jax
import jax.numpy as jnp
from jax import lax
from jax.experimental import pallas as pl
from jax.experimental.pallas import tpu as pltpu
from jax.experimental.pallas import tpu_sc as plsc

_K = 20
_FH = 240
_FW = 320


def _conv2d(x, w, b=None, pad=0):
    out = lax.conv_general_dilated(
        x, w, (1, 1), [(pad, pad), (pad, pad)],
        dimension_numbers=('NCHW', 'OIHW', 'NCHW'))
    if b is not None:
        out = out + b[None, :, None, None]
    return out


def _bn(x, g, b):
    m = jnp.mean(x, axis=(0, 2, 3), keepdims=True)
    v = jnp.var(x, axis=(0, 2, 3), keepdims=True)
    return (x - m) / jnp.sqrt(v + 1e-5) * g[None, :, None, None] + b[None, :, None, None]


def _conv_hwc(x, w, b=None, pad=0):
    out = lax.conv_general_dilated(
        x, w, (1, 1), [(pad, pad), (pad, pad)],
        dimension_numbers=('NHWC', 'OIHW', 'NHWC'))
    if b is not None:
        out = out + b[None, None, None, :]
    return out


def _bn_hwc(x, g, b):
    m = jnp.mean(x, axis=(0, 1, 2), keepdims=True)
    v = jnp.var(x, axis=(0, 1, 2), keepdims=True)
    return (x - m) / jnp.sqrt(v + 1e-5) * g[None, None, None, :] + b[None, None, None, :]


def _coord_att_hwc(x, w1, b1, g1, bb1, wh, bh, ww, bw):
    B, H, W, C = x.shape
    xh = jnp.mean(x, axis=2, keepdims=True)
    xw = jnp.mean(x, axis=1, keepdims=True)
    y = jnp.concatenate([xh, jnp.transpose(xw, (0, 2, 1, 3))], axis=1)
    y = _bn_hwc(_conv_hwc(y, w1, b1), g1, bb1)
    y = y * jnp.clip(y + 3.0, 0.0, 6.0) / 6.0
    yh = y[:, :H]
    yw = jnp.transpose(y[:, H:], (0, 2, 1, 3))
    ah = jax.nn.sigmoid(_conv_hwc(yh, wh, bh))
    aw = jax.nn.sigmoid(_conv_hwc(yw, ww, bw))
    return x * ah * aw


def _lrelu(x):
    return jnp.where(x >= 0, x, 0.2 * x)


def _coord_att(x, w1, b1, g1, bb1, wh, bh, ww, bw):
    B, C, H, W = x.shape
    xh = jnp.mean(x, axis=3, keepdims=True)
    xw = jnp.transpose(jnp.mean(x, axis=2, keepdims=True), (0, 1, 3, 2))
    y = jnp.concatenate([xh, xw], axis=2)
    y = _bn(_conv2d(y, w1, b1), g1, bb1)
    y = y * jnp.clip(y + 3.0, 0.0, 6.0) / 6.0
    yh = y[:, :, :H]
    yw = jnp.transpose(y[:, :, H:], (0, 1, 3, 2))
    ah = jax.nn.sigmoid(_conv2d(yh, wh, bh))
    aw = jax.nn.sigmoid(_conv2d(yw, ww, bw))
    return x * ah * aw


def _topk_body(k, TN, N, C, xt_ref, xf_ref, idx_ref, pd_ref):
    xt = xt_ref[0]
    xf = xf_ref[0]
    ip = lax.dot_general(xt.astype(jnp.bfloat16), xf.astype(jnp.bfloat16),
                         (((1,), (1,)), ((), ())),
                         preferred_element_type=jnp.float32)
    xx_t = jnp.sum(xt * xt, axis=1, keepdims=True)
    xx_f = jnp.sum(xf * xf, axis=1, keepdims=True).reshape(1, N)
    pd_ref[:] = (2.0 * ip - xx_t) - xx_f

    iota = lax.broadcasted_iota(jnp.int32, (TN, N), 1)
    liota = lax.broadcasted_iota(jnp.int32, (TN, k), 1)
    base = pl.program_id(0) * N
    acc = jnp.zeros((TN, k), jnp.int32)
    p = pd_ref[:]
    for t in range(k):
        jm = jnp.argmax(p, axis=1).astype(jnp.int32)[:, None]
        acc = jnp.where(liota == t, jm + base, acc)
        p = jnp.where(iota == jm, -jnp.inf, p)
    idx_ref[0] = acc


def _topk(x, k=_K, tn=512):
    B, N, C = x.shape
    NT = N // tn
    return pl.pallas_call(
        functools.partial(_topk_body, k, tn, N, C),
        grid=(B, NT),
        in_specs=[
            pl.BlockSpec((1, tn, C), lambda b, t: (b, t, 0)),
            pl.BlockSpec((1, N, C), lambda b, t: (b, 0, 0)),
        ],
        out_specs=pl.BlockSpec((1, tn, k), lambda b, t: (b, t, 0)),
        out_shape=jax.ShapeDtypeStruct((B, N, k), jnp.int32),
        scratch_shapes=[pltpu.VMEM((tn, N), jnp.float32)],
    )(x, x)


_SC_CHUNK = 128


def _sc_gather(x_flat, idx_flat):
    E = idx_flat.shape[0]
    C = x_flat.shape[1]
    info = plsc.get_sparse_core_info()
    nw = info.num_cores * info.num_subcores
    per_w = E // nw
    n_chunks = per_w // _SC_CHUNK
    mesh = plsc.VectorSubcoreMesh(core_axis_name="c", subcore_axis_name="s")

    @functools.partial(
        pl.kernel, mesh=mesh,
        out_type=jax.ShapeDtypeStruct((E, C), jnp.float32),
        scratch_types=[
            pltpu.VMEM((_SC_CHUNK,), jnp.int32),
            pltpu.VMEM((_SC_CHUNK, C), jnp.float32),
            pltpu.SemaphoreType.DMA,
        ],
    )
    def gk(x_hbm, idx_hbm, out_hbm, idx_v, rows_v, sem):
        wid = lax.axis_index("s") * info.num_cores + lax.axis_index("c")
        wbase = wid * per_w

        def body(c, _):
            off = wbase + c * _SC_CHUNK
            pltpu.sync_copy(idx_hbm.at[pl.ds(off, _SC_CHUNK)], idx_v)
            pltpu.async_copy(x_hbm.at[idx_v], rows_v, sem).wait()
            pltpu.sync_copy(rows_v, out_hbm.at[pl.ds(off, _SC_CHUNK)])
            return 0

        lax.fori_loop(0, n_chunks, body, 0)

    return gk(x_flat, idx_flat)


def _econv_body(k, TN, C, Cout, g_ref, xt_ref, w_ref, premax_ref, stats_ref):
    xt = xt_ref[0]
    g3 = g_ref[0][:, :C].reshape(TN, k, C)
    e = g3 - xt[:, None, :]
    xr = jnp.broadcast_to(xt[:, None, :], (TN, k, C))
    ev = jnp.concatenate([e, xr], axis=2).reshape(TN * k, 2 * C)
    y = lax.dot_general(ev.astype(jnp.bfloat16), w_ref[:].astype(jnp.bfloat16),
                        (((1,), (1,)), ((), ())),
                        preferred_element_type=jnp.float32)
    y3 = y.reshape(TN, k, Cout)
    premax_ref[0] = jnp.max(y3, axis=1)
    s1 = jnp.sum(y, axis=0, keepdims=True)
    s2 = jnp.sum(y * y, axis=0, keepdims=True)

    first = jnp.logical_and(pl.program_id(0) == 0, pl.program_id(1) == 0)

    @pl.when(first)
    def _():
        stats_ref[:] = jnp.zeros((8, Cout), jnp.float32)

    pad = jnp.zeros((6, Cout), jnp.float32)
    stats_ref[:] = stats_ref[:] + jnp.concatenate([s1, s2, pad], axis=0)


def _edgeconv(x, w, gamma, beta, k=_K, tn=512):
    B, N, C = x.shape
    Cout = w.shape[0]
    NT = N // tn

    idx = _topk(x, k=k, tn=1024)
    cp = max(128, C)
    xp = jnp.pad(x, ((0, 0), (0, 0), (0, cp - C))) if cp != C else x
    gath = _sc_gather(xp.reshape(B * N, cp), idx.reshape(-1))
    gath = gath.reshape(B, N * k, cp)

    premax, stats = pl.pallas_call(
        functools.partial(_econv_body, k, tn, C, Cout),
        grid=(B, NT),
        in_specs=[
            pl.BlockSpec((1, tn * k, cp), lambda b, t: (b, t, 0)),
            pl.BlockSpec((1, tn, C), lambda b, t: (b, t, 0)),
            pl.BlockSpec((Cout, 2 * C), lambda b, t: (0, 0)),
        ],
        out_specs=[
            pl.BlockSpec((1, tn, Cout), lambda b, t: (b, t, 0)),
            pl.BlockSpec((8, Cout), lambda b, t: (0, 0)),
        ],
        out_shape=[
            jax.ShapeDtypeStruct((B, N, Cout), jnp.float32),
            jax.ShapeDtypeStruct((8, Cout), jnp.float32),
        ],
    )(gath, x, w)

    cnt = B * N * k
    mean = stats[0] / cnt
    var = stats[1] / cnt - mean * mean
    out = (premax - mean[None, None, :]) / jnp.sqrt(var + 1e-5)[None, None, :] \
        * gamma[None, None, :] + beta[None, None, :]
    return jnp.where(out >= 0, out, 0.2 * out)


def _scatter_body(N, HW2, pidx_ref, x_ref, out_ref):
    out_ref[:] = jnp.zeros(out_ref.shape, out_ref.dtype)
    bb = pl.program_id(0)

    def body(i, _):
        p = pidx_ref[bb, i]
        out_ref[0, pl.ds(p, 1), :] = out_ref[0, pl.ds(p, 1), :] + x_ref[0, pl.ds(i, 1), :]
        return 0

    lax.fori_loop(0, N, body, 0)


def _scatter_add(x, pidx):
    B, N, C = x.shape
    HW = _FH * _FW
    HW2 = HW // 2
    z = jnp.zeros_like(x)
    lo = jnp.concatenate([x, z], axis=2)
    hi = jnp.concatenate([z, x], axis=2)
    x128 = jnp.where((pidx % 2 == 0)[:, :, None], lo, hi)
    r = pidx // 2
    out = pl.pallas_call(
        functools.partial(_scatter_body, N, HW2),
        grid=(B,),
        in_specs=[
            pl.BlockSpec(memory_space=pltpu.SMEM),
            pl.BlockSpec((1, N, 2 * C), lambda b: (b, 0, 0)),
        ],
        out_specs=pl.BlockSpec((1, HW2, 2 * C), lambda b: (b, 0, 0)),
        out_shape=jax.ShapeDtypeStruct((B, HW2, 2 * C), jnp.float32),
    )(r, x128)
    return out.reshape(B, HW, C)


def kernel(pc_xyzrgb, feat_s00, pre_w, pre_b, ca1_w, ca1_b, ca_g, ca_b2, cah_w, cah_b, caw_w, caw_b, c1_w, c1_g, c1_b, c2_w, c2_g, c2_b, c3_w, c3_g, c3_b, l4a_w, l4b_w, d5_w, d5_b, d5_g, d5_bb, d6_w, d6_b, d6_g, d6_bb, c7a_w, c7b_w):
    B = pc_xyzrgb.shape[0]
    N = pc_xyzrgb.shape[2]
    v = jnp.floor(pc_xyzrgb[:, 0, :] + 240.0).astype(jnp.int32).reshape(-1)
    u = jnp.floor(pc_xyzrgb[:, 1, :] + 320.0).astype(jnp.int32).reshape(-1)
    bi = jnp.repeat(jnp.arange(B, dtype=jnp.int32), N)
    pc_t = jnp.transpose(pc_xyzrgb, (0, 2, 1))

    fs = jnp.transpose(feat_s00, (0, 2, 3, 1))
    f = _conv_hwc(fs, pre_w, pre_b, pad=1)
    f = _coord_att_hwc(f, ca1_w, ca1_b, ca_g, ca_b2, cah_w, cah_b, caw_w, caw_b)
    feat_2d = f[bi, v, u].reshape(B, N, 24)

    x0 = jnp.concatenate([pc_t, feat_2d], axis=2)
    x1 = _edgeconv(x0, c1_w.reshape(64, 64), c1_g, c1_b)
    x2 = _edgeconv(x1, c2_w.reshape(64, 128), c2_g, c2_b)
    x3 = _edgeconv(x2, c3_w.reshape(128, 128), c3_g, c3_b)

    x = jnp.concatenate([x1, x2, x3], axis=2)
    x = jnp.matmul(jnp.matmul(x, l4a_w.T), l4b_w.T)

    v2 = v // 2
    u2 = u // 2
    pidx = (v2 * _FW + u2).reshape(B, N)
    fm = _scatter_add(x, pidx)
    fm = fm.reshape(B, _FH, _FW, 64)

    fm = jax.nn.softmax(fm, axis=3)
    fm = _lrelu(_bn_hwc(_conv_hwc(fm, d5_w, d5_b, pad=1), d5_g, d5_bb))
    fm = _lrelu(_bn_hwc(_conv_hwc(fm, d6_w, d6_b, pad=1), d6_g, d6_bb))
    fm = _conv_hwc(_conv_hwc(fm, c7a_w), c7b_w)
    fm = jnp.transpose(fm, (0, 3, 1, 2))
    return (fm, (bi, v2, u2))
